# baseline (speedup 1.0000x reference)
import jax
import jax.numpy as jnp
from jax import lax
from jax.experimental import pallas as pl
from jax.experimental.pallas import tpu as pltpu
from jax.experimental.pallas import tpu_sc as plsc

_B = 8
_D = 21
_H = 512
_W = 512
_S = _H * _W
_NW = 32
_CROWS = _B * _H // _NW
_K = 2048
_R = _K // _W
_NSUB = _CROWS // _R
_L = 16
_U = 4


def _sc_body(x_hbm, out_hbm,
             x0, x1, x2, x3, y0, y1,
             xs0, xs1, xs2, xs3, ys0, ys1):
    cid = lax.axis_index("c")
    sid = lax.axis_index("s")
    wid = sid * 2 + cid
    row0 = wid * _CROWS
    in_base = row0 * _W

    iota = lax.broadcasted_iota(jnp.int32, (_L,), 0)
    ones_v = jnp.ones((_L,), jnp.float32)
    zeros_v = jnp.zeros((_L,), jnp.float32)
    xbufs, xsems = (x0, x1, x2, x3), (xs0, xs1, xs2, xs3)
    ybufs, ysems = (y0, y1), (ys0, ys1)

    def xload(j, t):
        pltpu.async_copy(
            x_hbm.at[pl.ds(in_base + j * _K, _K)], xbufs[t], xsems[t]
        )

    def xwait(t):
        pltpu.make_async_copy(
            x_hbm.at[pl.ds(in_base, _K)], xbufs[t], xsems[t]
        ).wait()

    for yb in ybufs:
        def zrow(ch, _, yb=yb):
            for r in range(_R):
                def zcol(i, _, r=r):
                    base = i * (_L * _U)
                    for u in range(_U):
                        yb[ch, r, pl.ds(base + u * _L, _L)] = zeros_v
                    return 0
                lax.fori_loop(0, _W // (_L * _U), zcol, 0)
            return 0
        lax.fori_loop(0, _D, zrow, 0)

    for t in range(4):
        xload(t, t)

    def scatter_pass(yb, xb, val):
        def body(i, _):
            base = i * (_L * _U)
            for u in range(_U):
                xv = xb[pl.ds(base + u * _L, _L)]
                pos = base + u * _L + iota
                rowv = lax.shift_right_logical(pos, 9)
                colv = lax.bitwise_and(pos, _W - 1)
                plsc.store_scatter(yb, [xv, rowv, colv], val)
            return 0
        lax.fori_loop(0, _K // (_L * _U), body, 0)

    def drain(p):
        pltpu.make_async_copy(
            ybufs[p], out_hbm.at[0, :, pl.ds(0, _R), :], ysems[p]
        ).wait()

    def quad(qq, _):
        for t in range(4):
            p = t % 2
            j = 4 * qq + t
            yb = ybufs[p]

            def drain_restore(t=t, p=p):
                drain(p)
                scatter_pass(ybufs[p], xbufs[(t + 2) % 4], zeros_v)

            if t < 2:
                pl.when(qq > 0)(drain_restore)
            else:
                drain_restore()

            @pl.when(j + 2 < _NSUB)
            def _(t=t, j=j):
                xload(j + 2, (t + 2) % 4)

            xwait(t)
            scatter_pass(yb, xbufs[t], ones_v)
            grow = row0 + j * _R
            b = grow // _H
            r = grow - b * _H
            pltpu.async_copy(
                yb, out_hbm.at[b, :, pl.ds(r, _R), :], ysems[p]
            )
        return 0

    lax.fori_loop(0, _NSUB // 4, quad, 0)
    drain(0)
    drain(1)


def _sc_one_hot(x_flat):
    mesh = plsc.VectorSubcoreMesh(core_axis_name="c", subcore_axis_name="s")
    f = pl.kernel(
        _sc_body,
        out_type=jax.ShapeDtypeStruct((_B, _D, _H, _W), jnp.float32),
        mesh=mesh,
        scratch_types=[
            pltpu.VMEM((_K,), jnp.int32),
            pltpu.VMEM((_K,), jnp.int32),
            pltpu.VMEM((_K,), jnp.int32),
            pltpu.VMEM((_K,), jnp.int32),
            pltpu.VMEM((_D, _R, _W), jnp.float32),
            pltpu.VMEM((_D, _R, _W), jnp.float32),
            pltpu.SemaphoreType.DMA,
            pltpu.SemaphoreType.DMA,
            pltpu.SemaphoreType.DMA,
            pltpu.SemaphoreType.DMA,
            pltpu.SemaphoreType.DMA,
            pltpu.SemaphoreType.DMA,
        ],
        compiler_params=pltpu.CompilerParams(needs_layout_passes=False),
    )
    return f(x_flat)


def kernel(X_in, ones):
    del ones
    return _sc_one_hot(X_in.reshape(-1).astype(jnp.int32))


# speedup vs baseline: 1.0036x; 1.0036x over previous
import jax
import jax.numpy as jnp
from jax import lax
from jax.experimental import pallas as pl
from jax.experimental.pallas import tpu as pltpu
from jax.experimental.pallas import tpu_sc as plsc

_B = 8
_D = 21
_H = 512
_W = 512
_S = _H * _W
_NW = 32
_CROWS = _B * _H // _NW
_K = 2048
_R = _K // _W
_NSUB = _CROWS // _R
_L = 16
_U = 4


def _sc_body(x_hbm, out_hbm,
             x0, x1, x2, x3, y0, y1,
             xs0, xs1, xs2, xs3, ys0, ys1):
    cid = lax.axis_index("c")
    sid = lax.axis_index("s")
    wid = sid * 2 + cid
    row0 = wid * _CROWS
    in_base = row0 * _W

    iota = lax.broadcasted_iota(jnp.int32, (_L,), 0)
    ones_v = jnp.ones((_L,), jnp.float32)
    zeros_v = jnp.zeros((_L,), jnp.float32)
    xbufs, xsems = (x0, x1, x2, x3), (xs0, xs1, xs2, xs3)
    ybufs, ysems = (y0, y1), (ys0, ys1)

    def xload(j, t):
        pltpu.async_copy(
            x_hbm.at[pl.ds(in_base + j * _K, _K)], xbufs[t], xsems[t]
        )

    def xwait(t):
        pltpu.make_async_copy(
            x_hbm.at[pl.ds(in_base, _K)], xbufs[t], xsems[t]
        ).wait()

    for yb in ybufs:
        def zrow(ch, _, yb=yb):
            for r in range(_R):
                def zcol(i, _, r=r):
                    base = i * (_L * _U)
                    for u in range(_U):
                        yb[ch, r, pl.ds(base + u * _L, _L)] = zeros_v
                    return 0
                lax.fori_loop(0, _W // (_L * _U), zcol, 0)
            return 0
        lax.fori_loop(0, _D, zrow, 0)

    for t in range(2):
        xload(t, t)

    def scatter_pass(yb, xb, val):
        def body(i, _):
            base = i * (_L * _U)
            for u in range(_U):
                xv = xb[pl.ds(base + u * _L, _L)]
                pos = base + u * _L + iota
                rowv = lax.shift_right_logical(pos, 9)
                colv = lax.bitwise_and(pos, _W - 1)
                plsc.store_scatter(yb, [xv, rowv, colv], val)
            return 0
        lax.fori_loop(0, _K // (_L * _U), body, 0)

    def drain(p):
        pltpu.make_async_copy(
            ybufs[p], out_hbm.at[0, :, pl.ds(0, _R), :], ysems[p]
        ).wait()

    def quad(qq, _):
        for t in range(4):
            p = t % 2
            j = 4 * qq + t
            yb = ybufs[p]

            def drain_restore(t=t, p=p):
                drain(p)
                scatter_pass(ybufs[p], xbufs[(t + 2) % 4], zeros_v)

            if t < 2:
                pl.when(qq > 0)(drain_restore)
            else:
                drain_restore()

            @pl.when(j + 2 < _NSUB)
            def _(t=t, j=j):
                xload(j + 2, (t + 2) % 4)

            xwait(t)
            scatter_pass(yb, xbufs[t], ones_v)
            grow = row0 + j * _R
            b = grow // _H
            r = grow - b * _H
            pltpu.async_copy(
                yb, out_hbm.at[b, :, pl.ds(r, _R), :], ysems[p]
            )
        return 0

    lax.fori_loop(0, _NSUB // 4, quad, 0)
    drain(0)
    drain(1)


def _sc_one_hot(x_flat):
    mesh = plsc.VectorSubcoreMesh(core_axis_name="c", subcore_axis_name="s")
    f = pl.kernel(
        _sc_body,
        out_type=jax.ShapeDtypeStruct((_B, _D, _H, _W), jnp.float32),
        mesh=mesh,
        scratch_types=[
            pltpu.VMEM((_K,), jnp.int32),
            pltpu.VMEM((_K,), jnp.int32),
            pltpu.VMEM((_K,), jnp.int32),
            pltpu.VMEM((_K,), jnp.int32),
            pltpu.VMEM((_D, _R, _W), jnp.float32),
            pltpu.VMEM((_D, _R, _W), jnp.float32),
            pltpu.SemaphoreType.DMA,
            pltpu.SemaphoreType.DMA,
            pltpu.SemaphoreType.DMA,
            pltpu.SemaphoreType.DMA,
            pltpu.SemaphoreType.DMA,
            pltpu.SemaphoreType.DMA,
        ],
        compiler_params=pltpu.CompilerParams(needs_layout_passes=False),
    )
    return f(x_flat)


def kernel(X_in, ones):
    del ones
    return _sc_one_hot(X_in.reshape(-1).astype(jnp.int32))
